# Initial kernel scaffold; baseline (speedup 1.0000x reference)
import jax
import jax.numpy as jnp
from jax.experimental import pallas as pl


def kernel(user_ids, item_ids, user_table, item_table):
    raise NotImplementedError("write your pallas kernel here")


# speedup vs baseline: 1.2982x; 1.2982x over previous
import functools

import jax
import jax.numpy as jnp
from jax import lax
from jax.experimental import pallas as pl
from jax.experimental.pallas import tpu as pltpu
from jax.experimental.pallas import tpu_sc as plsc

EMBED = 64
CHUNK = 128


@functools.lru_cache(maxsize=None)
def _make_kernel(n_user_chunks, n_item_chunks):
    info = plsc.get_sparse_core_info()
    nw = info.num_cores * info.num_subcores
    nc = info.num_cores
    u_per_w = n_user_chunks // nw
    i_per_w = n_item_chunks // nw

    mesh = plsc.VectorSubcoreMesh(core_axis_name="c", subcore_axis_name="s")

    @functools.partial(
        pl.kernel,
        mesh=mesh,
        out_type=(
            jax.ShapeDtypeStruct((n_user_chunks * CHUNK, EMBED), jnp.float32),
            jax.ShapeDtypeStruct((n_item_chunks * CHUNK, EMBED), jnp.float32),
        ),
        scratch_types=[
            pltpu.VMEM((u_per_w, CHUNK), jnp.int32),
            pltpu.VMEM((i_per_w, CHUNK), jnp.int32),
            pltpu.VMEM((CHUNK, EMBED), jnp.float32),
            pltpu.SemaphoreType.DMA,
        ],
        compiler_params=pltpu.CompilerParams(use_tc_tiling_on_sc=False),
    )
    def sc_gather(user_ids, item_ids, user_table, item_table,
                  user_out, item_out, uidx_v, iidx_v, rows_v, sem):
        wid = lax.axis_index("s") * nc + lax.axis_index("c")
        ubase = wid * u_per_w
        ibase = wid * i_per_w
        pltpu.sync_copy(user_ids.at[pl.ds(ubase, u_per_w)], uidx_v)
        pltpu.sync_copy(item_ids.at[pl.ds(ibase, i_per_w)], iidx_v)

        def ubody(j, carry):
            pltpu.async_copy(user_table.at[uidx_v.at[j]], rows_v, sem).wait()
            pltpu.sync_copy(rows_v, user_out.at[pl.ds((ubase + j) * CHUNK, CHUNK)])
            return carry

        lax.fori_loop(0, u_per_w, ubody, 0)

        def ibody(j, carry):
            pltpu.async_copy(item_table.at[iidx_v.at[j]], rows_v, sem).wait()
            pltpu.sync_copy(rows_v, item_out.at[pl.ds((ibase + j) * CHUNK, CHUNK)])
            return carry

        lax.fori_loop(0, i_per_w, ibody, 0)

    return sc_gather


def kernel(user_ids, item_ids, user_table, item_table):
    b = user_ids.shape[0]
    bh, hist = item_ids.shape
    n_user_chunks = b // CHUNK
    n_item_chunks = (bh * hist) // CHUNK
    uids2 = user_ids.reshape(n_user_chunks, CHUNK)
    iids2 = item_ids.reshape(n_item_chunks, CHUNK)
    user_out, item_out = _make_kernel(n_user_chunks, n_item_chunks)(
        uids2, iids2, user_table, item_table)
    return user_out.reshape(b, EMBED), item_out.reshape(bh, hist, EMBED)


# speedup vs baseline: 1.4046x; 1.0820x over previous
import functools

import jax
import jax.numpy as jnp
from jax import lax
from jax.experimental import pallas as pl
from jax.experimental.pallas import tpu as pltpu
from jax.experimental.pallas import tpu_sc as plsc

EMBED = 64
CHUNK = 128
K = 4


@functools.lru_cache(maxsize=None)
def _make_kernel(n_user_chunks, n_item_chunks):
    info = plsc.get_sparse_core_info()
    nw = info.num_cores * info.num_subcores
    nc = info.num_cores
    u_per_w = n_user_chunks // nw
    i_per_w = n_item_chunks // nw
    n_super = i_per_w // K
    rows_per_super = K * CHUNK

    mesh = plsc.VectorSubcoreMesh(core_axis_name="c", subcore_axis_name="s")

    @functools.partial(
        pl.kernel,
        mesh=mesh,
        out_type=(
            jax.ShapeDtypeStruct((n_user_chunks * CHUNK, EMBED), jnp.float32),
            jax.ShapeDtypeStruct((n_item_chunks * CHUNK, EMBED), jnp.float32),
        ),
        scratch_types=[
            pltpu.VMEM((u_per_w, CHUNK), jnp.int32),
            pltpu.VMEM((i_per_w, CHUNK), jnp.int32),
            pltpu.VMEM((rows_per_super, EMBED), jnp.float32),
            pltpu.VMEM((rows_per_super, EMBED), jnp.float32),
            pltpu.SemaphoreType.DMA,
            pltpu.SemaphoreType.DMA,
            pltpu.SemaphoreType.DMA,
            pltpu.SemaphoreType.DMA,
        ],
        compiler_params=pltpu.CompilerParams(use_tc_tiling_on_sc=False),
    )
    def sc_gather(user_ids, item_ids, user_table, item_table,
                  user_out, item_out, uidx_v, iidx_v, buf_a, buf_b,
                  gsem_a, gsem_b, wsem_a, wsem_b):
        wid = lax.axis_index("s") * nc + lax.axis_index("c")
        ubase = wid * u_per_w
        ibase = wid * i_per_w
        item_row_base = ibase * CHUNK
        pltpu.sync_copy(user_ids.at[pl.ds(ubase, u_per_w)], uidx_v)
        pltpu.sync_copy(item_ids.at[pl.ds(ibase, i_per_w)], iidx_v)

        for b in range(u_per_w):
            pltpu.make_async_copy(
                user_table.at[uidx_v.at[b]],
                buf_a.at[pl.ds(b * CHUNK, CHUNK)], gsem_a).start()
        pltpu.make_async_copy(
            user_table.at[pl.ds(0, u_per_w * CHUNK)], buf_a, gsem_a).wait()
        pltpu.sync_copy(buf_a, user_out.at[pl.ds(ubase * CHUNK,
                                                 u_per_w * CHUNK)])

        def fire(buf, sem, i):
            for b in range(K):
                pltpu.make_async_copy(
                    item_table.at[iidx_v.at[i * K + b]],
                    buf.at[pl.ds(b * CHUNK, CHUNK)], sem).start()

        def drain_gathers(buf, sem):
            pltpu.make_async_copy(
                item_table.at[pl.ds(0, rows_per_super)], buf, sem).wait()

        def write(buf, sem, i):
            pltpu.make_async_copy(
                buf, item_out.at[pl.ds(item_row_base + i * rows_per_super,
                                       rows_per_super)], sem).start()

        def wait_write(buf, sem):
            pltpu.make_async_copy(
                buf, item_out.at[pl.ds(item_row_base, rows_per_super)],
                sem).wait()

        def phase(i, cur_buf, cur_g, cur_w, oth_buf, oth_g, oth_w,
                  first=False, fire_next=True):
            if not first:
                wait_write(oth_buf, oth_w)
            if fire_next:
                fire(oth_buf, oth_g, i + 1)
            drain_gathers(cur_buf, cur_g)
            write(cur_buf, cur_w, i)

        def phase_a(i, **kw):
            phase(i, buf_a, gsem_a, wsem_a, buf_b, gsem_b, wsem_b, **kw)

        def phase_b(i, **kw):
            phase(i, buf_b, gsem_b, wsem_b, buf_a, gsem_a, wsem_a, **kw)

        fire(buf_a, gsem_a, 0)
        phase_a(0, first=True)
        phase_b(1)

        def pair(p, carry):
            phase_a(2 * p)
            phase_b(2 * p + 1)
            return carry

        lax.fori_loop(1, n_super // 2 - 1, pair, 0)

        phase_a(n_super - 2)
        phase_b(n_super - 1, fire_next=False)
        wait_write(buf_b, wsem_b)

    return sc_gather


def kernel(user_ids, item_ids, user_table, item_table):
    b = user_ids.shape[0]
    bh, hist = item_ids.shape
    n_user_chunks = b // CHUNK
    n_item_chunks = (bh * hist) // CHUNK
    uids2 = user_ids.reshape(n_user_chunks, CHUNK)
    iids2 = item_ids.reshape(n_item_chunks, CHUNK)
    user_out, item_out = _make_kernel(n_user_chunks, n_item_chunks)(
        uids2, iids2, user_table, item_table)
    return user_out.reshape(b, EMBED), item_out.reshape(bh, hist, EMBED)
